# baseline (speedup 1.0000x reference)
import functools

import jax
import jax.numpy as jnp
from jax import lax
from jax.experimental import pallas as pl
from jax.experimental.pallas import tpu as pltpu
from jax.experimental.pallas import tpu_sc as plsc

B = 16384
D_ITEM, D_USER, D_CAT, D_AGE = 20, 20, 17, 7
D_OUT = D_ITEM + D_USER + D_CAT + D_AGE + 1

_info = plsc.get_sparse_core_info()
NC, NS, L = _info.num_cores, _info.num_subcores, _info.num_lanes
NW = NC * NS
BPW = B // NW

_mesh = plsc.VectorSubcoreMesh(core_axis_name="c", subcore_axis_name="s")


@functools.partial(
    pl.kernel,
    mesh=_mesh,
    compiler_params=pltpu.CompilerParams(use_tc_tiling_on_sc=False,
                                         needs_layout_passes=False),
    out_type=jax.ShapeDtypeStruct((B * D_OUT,), jnp.float32),
    scratch_types=[
        pltpu.VMEM((BPW,), jnp.int32),
        pltpu.VMEM((BPW,), jnp.int32),
        pltpu.VMEM((BPW,), jnp.int32),
        pltpu.VMEM((BPW,), jnp.int32),
        pltpu.VMEM((BPW,), jnp.float32),
        pltpu.VMEM((BPW, D_ITEM), jnp.float32),
        pltpu.VMEM((BPW, D_USER), jnp.float32),
        pltpu.VMEM((BPW, D_CAT), jnp.float32),
        pltpu.VMEM((BPW, D_AGE), jnp.float32),
        pltpu.VMEM((BPW * D_OUT,), jnp.float32),
        pltpu.SemaphoreType.DMA,
    ],
)
def _emb_kernel(item_hbm, user_hbm, cat_hbm, age_hbm, price_hbm,
                W_item_hbm, W_user_hbm, W_cat_hbm, W_age_hbm, out_hbm,
                idx_item, idx_user, idx_cat, idx_age, price_v,
                r_item, r_user, r_cat, r_age, block, sem):
    wid = lax.axis_index("s") * NC + lax.axis_index("c")
    base = wid * BPW

    pltpu.sync_copy(item_hbm.at[pl.ds(base, BPW)], idx_item)
    pltpu.sync_copy(user_hbm.at[pl.ds(base, BPW)], idx_user)
    pltpu.sync_copy(cat_hbm.at[pl.ds(base, BPW)], idx_cat)
    pltpu.sync_copy(age_hbm.at[pl.ds(base, BPW)], idx_age)
    pltpu.sync_copy(price_hbm.at[pl.ds(base, BPW)], price_v)

    copies = [
        pltpu.make_async_copy(W_item_hbm.at[idx_item], r_item, sem),
        pltpu.make_async_copy(W_user_hbm.at[idx_user], r_user, sem),
        pltpu.make_async_copy(W_cat_hbm.at[idx_cat], r_cat, sem),
        pltpu.make_async_copy(W_age_hbm.at[idx_age], r_age, sem),
    ]
    for c in copies:
        c.start()
    for c in copies:
        c.wait()

    lane = lax.iota(jnp.int32, L)

    def group_body(g, _):
        rows = g * L + lane
        dst0 = rows * D_OUT
        col = 0
        for src, width in ((r_item, D_ITEM), (r_user, D_USER),
                           (r_cat, D_CAT), (r_age, D_AGE)):
            for j in range(width):
                jv = jnp.full((L,), j, jnp.int32)
                v = plsc.load_gather(src, [rows, jv])
                plsc.store_scatter(block, [dst0 + col], v)
                col += 1
        v = plsc.load_gather(price_v, [rows])
        plsc.store_scatter(block, [dst0 + col], v)
        return _

    lax.fori_loop(0, BPW // L, group_body, 0)

    pltpu.sync_copy(block, out_hbm.at[pl.ds(base * D_OUT, BPW * D_OUT)])


def kernel(cat_item_id, cat_user_id, cat_category, disc_clip_age,
           norm_clip_price, W_item, W_user, W_cat, W_age):
    out = _emb_kernel(
        cat_item_id.astype(jnp.int32),
        cat_user_id.astype(jnp.int32),
        cat_category.astype(jnp.int32),
        disc_clip_age.astype(jnp.int32),
        norm_clip_price,
        W_item, W_user, W_cat, W_age,
    )
    return out.reshape(B, D_OUT)


# speedup vs baseline: 1.3581x; 1.3581x over previous
import functools

import jax
import jax.numpy as jnp
from jax import lax
from jax.experimental import pallas as pl
from jax.experimental.pallas import tpu as pltpu
from jax.experimental.pallas import tpu_sc as plsc

B = 16384
D_ITEM, D_USER, D_CAT, D_AGE = 20, 20, 17, 7
D_OUT = D_ITEM + D_USER + D_CAT + D_AGE + 1
V_ITEM, V_USER, V_CAT, V_AGE = 1000001, 1000001, 100001, 101

_info = plsc.get_sparse_core_info()
NC, NS, L = _info.num_cores, _info.num_subcores, _info.num_lanes
NW = NC * NS
BPW = B // NW
NG = BPW // L

_mesh = plsc.VectorSubcoreMesh(core_axis_name="c", subcore_axis_name="s")


@functools.partial(
    pl.kernel,
    mesh=_mesh,
    compiler_params=pltpu.CompilerParams(use_tc_tiling_on_sc=False,
                                         needs_layout_passes=False),
    out_type=jax.ShapeDtypeStruct((B * D_OUT,), jnp.float32),
    scratch_types=[
        pltpu.VMEM((BPW,), jnp.int32),
        pltpu.VMEM((BPW,), jnp.int32),
        pltpu.VMEM((BPW,), jnp.int32),
        pltpu.VMEM((BPW,), jnp.int32),
        pltpu.VMEM((BPW,), jnp.float32),
        pltpu.VMEM((V_AGE * D_AGE,), jnp.float32),
        pltpu.VMEM((BPW * D_ITEM,), jnp.int32),
        pltpu.VMEM((BPW * D_USER,), jnp.int32),
        pltpu.VMEM((BPW * D_CAT,), jnp.int32),
        pltpu.VMEM((BPW * D_ITEM,), jnp.float32),
        pltpu.VMEM((BPW * D_USER,), jnp.float32),
        pltpu.VMEM((BPW * D_CAT,), jnp.float32),
        pltpu.VMEM((BPW * D_OUT,), jnp.float32),
        pltpu.SemaphoreType.DMA,
    ],
)
def _emb_kernel(item_hbm, user_hbm, cat_hbm, age_hbm, price_hbm,
                W_item_hbm, W_user_hbm, W_cat_hbm, W_age_hbm, out_hbm,
                idx_item, idx_user, idx_cat, idx_age, price_v, age_tab,
                e_item, e_user, e_cat, r_item, r_user, r_cat, block, sem):
    wid = lax.axis_index("s") * NC + lax.axis_index("c")
    base = wid * BPW

    pltpu.sync_copy(item_hbm.at[pl.ds(base, BPW)], idx_item)
    pltpu.sync_copy(user_hbm.at[pl.ds(base, BPW)], idx_user)
    pltpu.sync_copy(cat_hbm.at[pl.ds(base, BPW)], idx_cat)
    pltpu.sync_copy(age_hbm.at[pl.ds(base, BPW)], idx_age)
    pltpu.sync_copy(price_hbm.at[pl.ds(base, BPW)], price_v)
    pltpu.sync_copy(W_age_hbm, age_tab)

    lane = lax.iota(jnp.int32, L)

    def build_body(g, _):
        rows = g * L + lane
        for idx, e_ref, width in ((idx_item, e_item, D_ITEM),
                                  (idx_user, e_user, D_USER),
                                  (idx_cat, e_cat, D_CAT)):
            v = idx[pl.ds(g * L, L)] * width
            dst = rows * width
            for j in range(width):
                plsc.store_scatter(e_ref, [dst + j], v + j)
        return _

    lax.fori_loop(0, NG, build_body, 0)

    copies = [
        pltpu.make_async_copy(W_item_hbm.at[e_item], r_item, sem),
        pltpu.make_async_copy(W_user_hbm.at[e_user], r_user, sem),
        pltpu.make_async_copy(W_cat_hbm.at[e_cat], r_cat, sem),
    ]
    for c in copies:
        c.start()
    for c in copies:
        c.wait()

    def group_body(g, _):
        rows = g * L + lane
        dst0 = rows * D_OUT
        col = 0
        for src, width in ((r_item, D_ITEM), (r_user, D_USER),
                           (r_cat, D_CAT)):
            s0 = rows * width
            for j in range(width):
                v = plsc.load_gather(src, [s0 + j])
                plsc.store_scatter(block, [dst0 + col], v)
                col += 1
        a0 = idx_age[pl.ds(g * L, L)] * D_AGE
        for j in range(D_AGE):
            v = plsc.load_gather(age_tab, [a0 + j])
            plsc.store_scatter(block, [dst0 + col], v)
            col += 1
        v = plsc.load_gather(price_v, [rows])
        plsc.store_scatter(block, [dst0 + col], v)
        return _

    lax.fori_loop(0, NG, group_body, 0)

    pltpu.sync_copy(block, out_hbm.at[pl.ds(base * D_OUT, BPW * D_OUT)])


def kernel(cat_item_id, cat_user_id, cat_category, disc_clip_age,
           norm_clip_price, W_item, W_user, W_cat, W_age):
    out = _emb_kernel(
        cat_item_id.astype(jnp.int32),
        cat_user_id.astype(jnp.int32),
        cat_category.astype(jnp.int32),
        disc_clip_age.astype(jnp.int32),
        norm_clip_price,
        W_item.reshape(-1), W_user.reshape(-1),
        W_cat.reshape(-1), W_age.reshape(-1),
    )
    return out.reshape(B, D_OUT)
